# baseline (speedup 1.0000x reference)
import jax
import jax.numpy as jnp
from jax.experimental import pallas as pl
from jax.experimental.pallas import tpu as pltpu

_N = 10000
_F = 128
_BM_SUP = 2000
_BM = 400


def _support_body(x_ref, w_ref, out_ref):
    s = jax.lax.dot_general(
        x_ref[...], w_ref[...], (((1,), (0,)), ((), ())),
        preferred_element_type=jnp.float32,
        precision=jax.lax.Precision.HIGHEST)
    out_ref[...] = s.astype(jnp.bfloat16)


def _spmm_body(adj_ref, sup_ref, out_ref):
    a = adj_ref[...].astype(jnp.bfloat16)
    acc = jax.lax.dot_general(
        a, sup_ref[...], (((1,), (0,)), ((), ())),
        preferred_element_type=jnp.float32)
    out_ref[...] = jnp.maximum(acc, 0.0)


def kernel(adj, x_input, weight):
    support = pl.pallas_call(
        _support_body,
        grid=(_N // _BM_SUP,),
        in_specs=[pl.BlockSpec((_BM_SUP, _F), lambda i: (i, 0)),
                  pl.BlockSpec((_F, _F), lambda i: (0, 0))],
        out_specs=pl.BlockSpec((_BM_SUP, _F), lambda i: (i, 0)),
        out_shape=jax.ShapeDtypeStruct((_N, _F), jnp.bfloat16),
        compiler_params=pltpu.CompilerParams(
            dimension_semantics=("arbitrary",)),
    )(x_input, weight)

    out = pl.pallas_call(
        _spmm_body,
        grid=(_N // _BM,),
        in_specs=[pl.BlockSpec((_BM, _N), lambda i: (i, 0)),
                  pl.BlockSpec((_N, _F), lambda i: (0, 0))],
        out_specs=pl.BlockSpec((_BM, _F), lambda i: (i, 0)),
        out_shape=jax.ShapeDtypeStruct((_N, _F), jnp.float32),
        compiler_params=pltpu.CompilerParams(
            dimension_semantics=("parallel",)),
    )(adj, support)
    return out


# speedup vs baseline: 1.0324x; 1.0324x over previous
import jax
import jax.numpy as jnp
from jax.experimental import pallas as pl
from jax.experimental.pallas import tpu as pltpu

_N = 10000
_F = 128
_BM = 200


def _fused_body(adj_ref, x_ref, w_ref, out_ref, sup_ref):
    @pl.when(pl.program_id(0) == 0)
    def _():
        s = jax.lax.dot_general(
            x_ref[...], w_ref[...], (((1,), (0,)), ((), ())),
            preferred_element_type=jnp.float32,
            precision=jax.lax.Precision.HIGHEST)
        sup_ref[...] = s.astype(jnp.bfloat16)

    a = adj_ref[...].astype(jnp.bfloat16)
    acc = jax.lax.dot_general(
        a, sup_ref[...], (((1,), (0,)), ((), ())),
        preferred_element_type=jnp.float32)
    out_ref[...] = jnp.maximum(acc, 0.0)


def kernel(adj, x_input, weight):
    return pl.pallas_call(
        _fused_body,
        grid=(_N // _BM,),
        in_specs=[pl.BlockSpec((_BM, _N), lambda i: (i, 0)),
                  pl.BlockSpec((_N, _F), lambda i: (0, 0)),
                  pl.BlockSpec((_F, _F), lambda i: (0, 0))],
        out_specs=pl.BlockSpec((_BM, _F), lambda i: (i, 0)),
        out_shape=jax.ShapeDtypeStruct((_N, _F), jnp.float32),
        scratch_shapes=[pltpu.VMEM((_N, _F), jnp.bfloat16)],
        compiler_params=pltpu.CompilerParams(
            dimension_semantics=("arbitrary",)),
    )(adj, x_input, weight)


# speedup vs baseline: 1.0658x; 1.0323x over previous
import jax
import jax.numpy as jnp
from jax.experimental import pallas as pl
from jax.experimental.pallas import tpu as pltpu

_N = 10000
_F = 128
_BM = 80
_NBUF = 5
_STEPS = _N // _BM


def _body(adj_hbm, x_ref, w_ref, out_ref, buf_ref, sem, sup_ref):
    sup_ref[...] = jax.lax.dot_general(
        x_ref[...], w_ref[...], (((1,), (0,)), ((), ())),
        preferred_element_type=jnp.float32,
        precision=jax.lax.Precision.HIGHEST)

    def _start(step, slot):
        pltpu.make_async_copy(
            adj_hbm.at[pl.ds(step * _BM, _BM), :],
            buf_ref.at[slot],
            sem.at[slot],
        ).start()

    for slot in range(_NBUF):
        _start(slot, slot)

    def _round(b, carry):
        for slot in range(_NBUF):
            i = b * _NBUF + slot
            pltpu.make_async_copy(
                adj_hbm.at[pl.ds(i * _BM, _BM), :],
                buf_ref.at[slot],
                sem.at[slot],
            ).wait()
            acc = jax.lax.dot_general(
                buf_ref[slot], sup_ref[...], (((1,), (0,)), ((), ())),
                preferred_element_type=jnp.float32)
            out_ref[pl.ds(i * _BM, _BM), :] = jnp.maximum(acc, 0.0)

            @pl.when(i + _NBUF < _STEPS)
            def _():
                _start(i + _NBUF, slot)

        return carry

    jax.lax.fori_loop(0, _STEPS // _NBUF, _round, 0)


def kernel(adj, x_input, weight):
    return pl.pallas_call(
        _body,
        in_specs=[pl.BlockSpec(memory_space=pl.ANY),
                  pl.BlockSpec((_N, _F), lambda: (0, 0)),
                  pl.BlockSpec((_F, _F), lambda: (0, 0))],
        out_specs=pl.BlockSpec((_N, _F), lambda: (0, 0)),
        out_shape=jax.ShapeDtypeStruct((_N, _F), jnp.float32),
        scratch_shapes=[
            pltpu.VMEM((_NBUF, _BM, _N), jnp.float32),
            pltpu.SemaphoreType.DMA((_NBUF,)),
            pltpu.VMEM((_N, _F), jnp.float32),
        ],
        compiler_params=pltpu.CompilerParams(
            dimension_semantics=()),
    )(adj, x_input, weight)
